# baseline (speedup 1.0000x reference)
import jax
import jax.numpy as jnp
from jax.experimental import pallas as pl
from jax.experimental.pallas import tpu as pltpu


def kernel(x, adj, W1, b1, Wmu, bmu, Wlv, blv):
    n, d = x.shape
    h_dim = W1.shape[1]
    e = Wmu.shape[1]

    bm = 400
    nb = n // bm

    def fused_kernel(x_ref, adj_ref, W1_ref, b1_ref, Wmu_ref, bmu_ref,
                     Wlv_ref, blv_ref, z_ref, mu_ref, lv_ref, s1_ref, hm_ref):
        p = pl.program_id(0)
        i = pl.program_id(1)

        @pl.when((p == 0) & (i == 0))
        def _():
            s1_ref[...] = jnp.dot(x_ref[...], W1_ref[...],
                                  preferred_element_type=jnp.float32)

        @pl.when(p == 0)
        def _():
            h = jnp.dot(adj_ref[...], s1_ref[...],
                        preferred_element_type=jnp.float32)
            h = jnp.maximum(h + b1_ref[...], 0.0)
            rows = pl.ds(i * bm, bm)
            hm_ref[rows, :e] = jnp.dot(h, Wmu_ref[...],
                                       preferred_element_type=jnp.float32)
            hm_ref[rows, e:] = jnp.dot(h, Wlv_ref[...],
                                       preferred_element_type=jnp.float32)

        @pl.when(p == 1)
        def _():
            out = jnp.dot(adj_ref[...], hm_ref[...],
                          preferred_element_type=jnp.float32)
            mu = out[:, :e] + bmu_ref[...]
            z_ref[...] = mu
            mu_ref[...] = mu
            lv_ref[...] = out[:, e:] + blv_ref[...]

    out_spec = pl.BlockSpec((bm, e), lambda p, i: (p * i, 0))
    small = lambda shape: pl.BlockSpec(shape, lambda p, i: (0, 0))
    z, mu, logvar = pl.pallas_call(
        fused_kernel,
        grid=(2, nb),
        in_specs=[
            small((n, d)),
            pl.BlockSpec((bm, n), lambda p, i: (i, 0)),
            small((d, h_dim)),
            small((1, h_dim)),
            small((h_dim, e)),
            small((1, e)),
            small((h_dim, e)),
            small((1, e)),
        ],
        out_specs=[out_spec, out_spec, out_spec],
        out_shape=[jax.ShapeDtypeStruct((n, e), jnp.float32)] * 3,
        scratch_shapes=[
            pltpu.VMEM((n, h_dim), jnp.float32),
            pltpu.VMEM((n, 2 * e), jnp.float32),
        ],
    )(x, adj, W1, b1[None, :], Wmu, bmu[None, :], Wlv, blv[None, :])

    return (z, mu, logvar)
